# baseline (speedup 1.0000x reference)
import functools

import jax
import jax.numpy as jnp
from jax import lax
from jax.experimental import pallas as pl
from jax.experimental.pallas import tpu as pltpu
from jax.experimental.pallas import tpu_sc as plsc

_NUM_CORES = 2
_NUM_SUBCORES = 16
_NUM_WORKERS = _NUM_CORES * _NUM_SUBCORES


@jax.jit
def kernel(c, table):
    B, = c.shape
    V, D = table.shape
    assert B % _NUM_WORKERS == 0
    b_per_w = B // _NUM_WORKERS

    n_chunks = 4
    assert b_per_w % n_chunks == 0
    chunk = b_per_w // n_chunks

    pair_w = 2 * D
    table_pairs = table.reshape(V // 2, pair_w)

    mesh = plsc.VectorSubcoreMesh(core_axis_name="c", subcore_axis_name="s")

    @functools.partial(
        pl.kernel,
        mesh=mesh,
        out_type=jax.ShapeDtypeStruct((B, pair_w), table.dtype),
        scratch_types=[
            pltpu.VMEM((b_per_w,), jnp.int32),
            pltpu.VMEM((b_per_w,), jnp.int32),
            [pltpu.VMEM((chunk, pair_w), table.dtype) for _ in range(n_chunks)],
            [pltpu.SemaphoreType.DMA for _ in range(n_chunks)],
            pltpu.SemaphoreType.DMA,
        ],
    )
    def gather_kernel(idx_hbm, table_hbm, out_hbm, idx_v, idx_p, rows,
                      gsems, wsem):
        wid = lax.axis_index("s") * _NUM_CORES + lax.axis_index("c")
        base = wid * b_per_w
        pltpu.sync_copy(idx_hbm.at[pl.ds(base, b_per_w)], idx_v)
        for j in range(b_per_w // 16):
            cvec = idx_v[pl.ds(j * 16, 16)]
            idx_p[pl.ds(j * 16, 16)] = lax.shift_right_logical(cvec, 1)
        copies = [
            pltpu.async_copy(
                table_hbm.at[idx_p.at[pl.ds(g * chunk, chunk)]],
                rows[g],
                gsems[g],
            )
            for g in range(n_chunks)
        ]
        writes = []
        for g in range(n_chunks):
            copies[g].wait()
            writes.append(
                pltpu.async_copy(
                    rows[g], out_hbm.at[pl.ds(base + g * chunk, chunk)], wsem
                )
            )
        for w in writes:
            w.wait()

    pairs = gather_kernel(c.astype(jnp.int32), table_pairs)
    odd = (c & 1).astype(jnp.bool_)[:, None]
    return jnp.where(odd, pairs[:, D:], pairs[:, :D])


# speedup vs baseline: 1.2508x; 1.2508x over previous
import functools

import jax
import jax.numpy as jnp
from jax import lax
from jax.experimental import pallas as pl
from jax.experimental.pallas import tpu as pltpu
from jax.experimental.pallas import tpu_sc as plsc

_NUM_CORES = 2
_NUM_SUBCORES = 16
_NUM_WORKERS = _NUM_CORES * _NUM_SUBCORES


@jax.jit
def kernel(c, table):
    B, = c.shape
    V, D = table.shape
    assert B % _NUM_WORKERS == 0
    b_per_w = B // _NUM_WORKERS

    n_chunks = 4
    assert b_per_w % n_chunks == 0
    chunk = b_per_w // n_chunks

    pair_w = 2 * D
    table_pairs = jnp.pad(table, ((0, 0), (0, pair_w - D)))

    mesh = plsc.VectorSubcoreMesh(core_axis_name="c", subcore_axis_name="s")

    @functools.partial(
        pl.kernel,
        mesh=mesh,
        out_type=jax.ShapeDtypeStruct((B, pair_w), table.dtype),
        scratch_types=[
            pltpu.VMEM((b_per_w,), jnp.int32),
            [pltpu.VMEM((chunk, pair_w), table.dtype) for _ in range(n_chunks)],
            [pltpu.SemaphoreType.DMA for _ in range(n_chunks)],
            pltpu.SemaphoreType.DMA,
        ],
    )
    def gather_kernel(idx_hbm, table_hbm, out_hbm, idx_v, rows,
                      gsems, wsem):
        wid = lax.axis_index("s") * _NUM_CORES + lax.axis_index("c")
        base = wid * b_per_w
        pltpu.sync_copy(idx_hbm.at[pl.ds(base, b_per_w)], idx_v)
        copies = [
            pltpu.async_copy(
                table_hbm.at[idx_v.at[pl.ds(g * chunk, chunk)]],
                rows[g],
                gsems[g],
            )
            for g in range(n_chunks)
        ]
        writes = []
        for g in range(n_chunks):
            copies[g].wait()
            writes.append(
                pltpu.async_copy(
                    rows[g], out_hbm.at[pl.ds(base + g * chunk, chunk)], wsem
                )
            )
        for w in writes:
            w.wait()

    padded = gather_kernel(c.astype(jnp.int32), table_pairs)
    return padded[:, :D]
